# baseline (speedup 1.0000x reference)
import functools

import jax
import jax.numpy as jnp
from jax import lax
from jax.experimental import pallas as pl
from jax.experimental.pallas import tpu as pltpu
from jax.experimental.pallas import tpu_sc as plsc

_BATCH = 4
_SEQ = 4096
_D = 1024

_NC = 2
_NS = 16
_NW = _NC * _NS

_SROWS = _SEQ // _NW
_CH = 16
_CHF = _CH * _D
_NCHUNK = _SROWS // _CH


def _sc_body(x_hbm, p_hbm, o_hbm, posbuf, iobuf, sem):
    wid = lax.axis_index("s") * _NC + lax.axis_index("c")
    base = wid * _SROWS * _D

    def chunk_body(j, carry):
        off = base + j * _CHF
        pltpu.sync_copy(p_hbm.at[pl.ds(off, _CHF)], posbuf)

        def batch_body(b, carry2):
            xoff = b * (_SEQ * _D) + off
            pltpu.sync_copy(x_hbm.at[pl.ds(xoff, _CHF)], iobuf)

            def add_body(i, carry3):
                sl = pl.ds(i * 16, 16)
                iobuf[sl] = iobuf[sl] + posbuf[sl]
                return carry3

            lax.fori_loop(0, _CHF // 16, add_body, 0)
            pltpu.sync_copy(iobuf, o_hbm.at[pl.ds(xoff, _CHF)])
            return carry2

        lax.fori_loop(0, _BATCH, batch_body, 0)
        return carry

    lax.fori_loop(0, _NCHUNK, chunk_body, 0)


_sc_add = functools.partial(
    pl.kernel,
    mesh=plsc.VectorSubcoreMesh(core_axis_name="c", subcore_axis_name="s"),
    out_type=jax.ShapeDtypeStruct((_BATCH * _SEQ * _D,), jnp.float32),
    scratch_types=[
        pltpu.VMEM((_CHF,), jnp.float32),
        pltpu.VMEM((_CHF,), jnp.float32),
        pltpu.SemaphoreType.DMA,
    ],
)(_sc_body)


def kernel(inputs, pos_table):
    out = _sc_add(inputs.reshape(-1), pos_table.reshape(-1))
    return out.reshape(inputs.shape)


# speedup vs baseline: 1.7694x; 1.7694x over previous
import functools

import jax
import jax.numpy as jnp
from jax import lax
from jax.experimental import pallas as pl
from jax.experimental.pallas import tpu as pltpu
from jax.experimental.pallas import tpu_sc as plsc

_BATCH = 4
_SEQ = 4096
_D = 1024

_NC = 2
_NS = 16
_NW = _NC * _NS

_SROWS = _SEQ // _NW
_CH = 16
_CHF = _CH * _D
_NCHUNK = _SROWS // _CH
_BSTRIDE = _SEQ * _D


def _sc_body(x_hbm, p_hbm, o_hbm,
             posbuf, in0, in1, out0, out1,
             isem0, isem1, osem0, osem1):
    wid = lax.axis_index("s") * _NC + lax.axis_index("c")
    base = wid * _SROWS * _D

    ins = (in0, in1)
    outs = (out0, out1)
    isems = (isem0, isem1)
    osems = (osem0, osem1)

    def xoff(j, b):
        return b * _BSTRIDE + base + j * _CHF

    pltpu.async_copy(x_hbm.at[pl.ds(xoff(0, 0), _CHF)], in0, isem0)

    def chunk(j, carry):
        pltpu.sync_copy(p_hbm.at[pl.ds(base + j * _CHF, _CHF)], posbuf)
        for b in range(_BATCH):
            slot = b % 2
            if b < _BATCH - 1:
                pltpu.async_copy(
                    x_hbm.at[pl.ds(xoff(j, b + 1), _CHF)],
                    ins[1 - slot], isems[1 - slot])
            else:
                @pl.when(j + 1 < _NCHUNK)
                def _():
                    pltpu.async_copy(
                        x_hbm.at[pl.ds(xoff(j + 1, 0), _CHF)],
                        ins[1 - slot], isems[1 - slot])
            pltpu.make_async_copy(
                x_hbm.at[pl.ds(0, _CHF)], ins[slot], isems[slot]).wait()
            if b >= 2:
                pltpu.make_async_copy(
                    outs[slot], o_hbm.at[pl.ds(0, _CHF)], osems[slot]).wait()
            else:
                @pl.when(j > 0)
                def _():
                    pltpu.make_async_copy(
                        outs[slot], o_hbm.at[pl.ds(0, _CHF)],
                        osems[slot]).wait()

            def add_body(i, c):
                off = i * 128
                for u in range(8):
                    sl = pl.ds(off + u * 16, 16)
                    outs[slot][sl] = ins[slot][sl] + posbuf[sl]
                return c

            lax.fori_loop(0, _CHF // 128, add_body, 0)
            pltpu.async_copy(
                outs[slot], o_hbm.at[pl.ds(xoff(j, b), _CHF)], osems[slot])
        return carry

    lax.fori_loop(0, _NCHUNK, chunk, 0)
    pltpu.make_async_copy(out0, o_hbm.at[pl.ds(0, _CHF)], osem0).wait()
    pltpu.make_async_copy(out1, o_hbm.at[pl.ds(0, _CHF)], osem1).wait()


_sc_add = functools.partial(
    pl.kernel,
    mesh=plsc.VectorSubcoreMesh(core_axis_name="c", subcore_axis_name="s"),
    out_type=jax.ShapeDtypeStruct((_BATCH * _SEQ * _D,), jnp.float32),
    scratch_types=[
        pltpu.VMEM((_CHF,), jnp.float32),
        pltpu.VMEM((_CHF,), jnp.float32),
        pltpu.VMEM((_CHF,), jnp.float32),
        pltpu.VMEM((_CHF,), jnp.float32),
        pltpu.VMEM((_CHF,), jnp.float32),
        pltpu.SemaphoreType.DMA,
        pltpu.SemaphoreType.DMA,
        pltpu.SemaphoreType.DMA,
        pltpu.SemaphoreType.DMA,
    ],
)(_sc_body)


def kernel(inputs, pos_table):
    out = _sc_add(inputs.reshape(-1), pos_table.reshape(-1))
    return out.reshape(inputs.shape)
